# baseline (speedup 1.0000x reference)
import functools

import jax
import jax.numpy as jnp
from jax import lax
from jax.experimental import pallas as pl
from jax.experimental.pallas import tpu as pltpu
from jax.experimental.pallas import tpu_sc as plsc

N = 10000
E = 20000
NNZ = 320000
F = 128
CW = 32
NQ = 4
NCLS = 16

NC = 2
NS = 16
CH = 128
CHUNKS = 157
EPS = CH * CHUNKS
NNZ_PAD = NS * EPS
PAD = NNZ_PAD - NNZ


def _make_sc_stage(T, S):
    rows_s = (S // NS) & ~7
    last_rows = S - (NS - 1) * rows_s

    mesh = plsc.VectorSubcoreMesh(core_axis_name="c", subcore_axis_name="s")

    @functools.partial(
        pl.kernel,
        mesh=mesh,
        compiler_params=pltpu.CompilerParams(use_tc_tiling_on_sc=False),
        out_type=jax.ShapeDtypeStruct((NQ * S, CW), jnp.float32),
        scratch_types=[
            pltpu.VMEM((CHUNKS, CH), jnp.int32),
            pltpu.VMEM((CHUNKS, CH), jnp.int32),
            pltpu.VMEM((CHUNKS, CH), jnp.float32),
            pltpu.VMEM((CH, CW), jnp.float32),
            pltpu.VMEM((CH, CW), jnp.float32),
            pltpu.VMEM((CH, CW), jnp.float32),
            pltpu.VMEM((CH, CW), jnp.float32),
            pltpu.VMEM((16, CW), jnp.float32),
            pltpu.VMEM_SHARED((S, CW), jnp.float32),
            pltpu.SemaphoreType.DMA,
            pltpu.SemaphoreType.DMA,
            pltpu.SemaphoreType.DMA,
            pltpu.SemaphoreType.DMA,
            pltpu.SemaphoreType.DMA,
            pltpu.SemaphoreType.DMA,
            pltpu.SemaphoreType.DMA,
            pltpu.SemaphoreType.DMA,
        ],
    )
    def stage(table, gidx, sidx, w, out, gidx_v, sidx_v, w_v, rows0, rows1,
              rows2, rows3, zbuf, acc, g0, g1, g2, g3, s0, s1, s2, s3):
        c = lax.axis_index("c")
        s = lax.axis_index("s")

        pltpu.sync_copy(gidx.at[s], gidx_v)
        pltpu.sync_copy(sidx.at[s], sidx_v)
        pltpu.sync_copy(w.at[s], w_v)

        for r in range(16):
            for u in range(CW // 16):
                zbuf[r, pl.ds(u * 16, 16)] = jnp.zeros((16,), jnp.float32)
        base = s * rows_s
        n16 = jnp.where(s == NS - 1, last_rows // 16, rows_s // 16)

        slots = [(rows0, g0, s0), (rows1, g1, s1), (rows2, g2, s2),
                 (rows3, g3, s3)]

        def start_gather(i, slot):
            pltpu.async_copy(table.at[gidx_v.at[i]], slot[0], slot[1])

        def wait_gather(i, slot):
            pltpu.make_async_copy(table.at[gidx_v.at[i]], slot[0],
                                  slot[1]).wait()

        def start_scatter(i, slot):
            pltpu.async_copy(slot[0], acc.at[sidx_v.at[i]], slot[2], add=True)

        def wait_scatter(i, slot):
            pltpu.make_async_copy(slot[0], acc.at[sidx_v.at[i]],
                                  slot[2]).wait()

        def scale(i, slot):
            rows_buf = slot[0]

            def scale_body(g, carry2):
                wv = w_v[i, pl.ds(g * 16, 16)]
                for e in range(16):
                    ws = wv[e]
                    k = g * 16 + e
                    for u in range(CW // 16):
                        sl = pl.ds(u * 16, 16)
                        rows_buf[k, sl] = rows_buf[k, sl] * ws
                return carry2

            lax.fori_loop(0, CH // 16, scale_body, 0, unroll=2)

        for p in range(2):
            q = 2 * c + p
            delta = (2 * c * T) if p == 0 else T

            def off_body(i, carry):
                for j in range(CH // 16):
                    sl = pl.ds(j * 16, 16)
                    gidx_v[i, sl] = gidx_v[i, sl] + delta
                return carry

            lax.fori_loop(0, CHUNKS, off_body, 0)

            def z_body(z, carry):
                pltpu.sync_copy(zbuf, acc.at[pl.ds(base + z * 16, 16)])
                return carry

            lax.fori_loop(0, n16, z_body, 0)
            plsc.subcore_barrier()

            start_gather(0, slots[0])
            start_gather(1, slots[1])
            for i0 in range(2):
                wait_gather(i0, slots[i0])
                scale(i0, slots[i0])
                start_gather(i0 + 2, slots[i0 + 2])
                start_scatter(i0, slots[i0])

            def quad_body(jj, carry):
                for k in range(4):
                    i = 2 + 4 * jj + k
                    sl = slots[(2 + k) % 4]
                    wait_gather(i, sl)
                    scale(i, sl)
                    wait_scatter(i - 2, slots[k])
                    start_gather(i + 2, slots[k])
                    start_scatter(i, sl)
                return carry

            lax.fori_loop(0, (CHUNKS - 5) // 4, quad_body, 0)
            i_e = CHUNKS - 3
            wait_gather(i_e, slots[2])
            scale(i_e, slots[2])
            wait_scatter(i_e - 2, slots[0])
            start_gather(i_e + 2, slots[0])
            start_scatter(i_e, slots[2])
            wait_gather(i_e + 1, slots[3])
            scale(i_e + 1, slots[3])
            wait_scatter(i_e - 1, slots[1])
            start_scatter(i_e + 1, slots[3])
            wait_gather(i_e + 2, slots[0])
            scale(i_e + 2, slots[0])
            wait_scatter(i_e, slots[2])
            start_scatter(i_e + 2, slots[0])
            wait_scatter(i_e + 1, slots[3])
            wait_scatter(i_e + 2, slots[0])
            plsc.subcore_barrier()

            def wb_body(z, carry):
                r0 = base + z * 16
                pltpu.sync_copy(acc.at[pl.ds(r0, 16)],
                                out.at[pl.ds(q * S + r0, 16)])
                return carry

            lax.fori_loop(0, n16, wb_body, 0)
            plsc.subcore_barrier()

    return stage


_raw_stage_n2e = _make_sc_stage(N, E)
_raw_stage_e2n = _make_sc_stage(E, N)


def _hbm(x):
    return pltpu.with_memory_space_constraint(x, pltpu.MemorySpace.HBM)


def _stage_n2e(table, gidx, sidx, w):
    return _raw_stage_n2e(_hbm(table), _hbm(gidx), _hbm(sidx), _hbm(w))


def _stage_e2n(table, gidx, sidx, w):
    return _raw_stage_e2n(_hbm(table), _hbm(gidx), _hbm(sidx), _hbm(w))


_RB = 1000


def _split_w(W):
    k = W.shape[1] // NQ
    return jnp.stack([W[:, i * k:(i + 1) * k] for i in range(NQ)])


def _split_b(b):
    return b.reshape(NQ, 1, b.shape[0] // NQ)


def _mm1_body(x_ref, w_ref, b_ref, o_ref):
    o_ref[...] = (
        jnp.dot(x_ref[...], w_ref[0], preferred_element_type=jnp.float32)
        + b_ref[0, 0]
    )


def _mm1(x, W, b):
    nb = N // _RB
    return pl.pallas_call(
        _mm1_body,
        grid=(NQ, nb),
        in_specs=[
            pl.BlockSpec((_RB, F), lambda q, r: (r, 0)),
            pl.BlockSpec((1, F, CW), lambda q, r: (q, 0, 0)),
            pl.BlockSpec((1, 1, CW), lambda q, r: (q, 0, 0)),
        ],
        out_specs=pl.BlockSpec((_RB, CW), lambda q, r: (q * nb + r, 0)),
        out_shape=jax.ShapeDtypeStruct((NQ * N, CW), jnp.float32),
    )(x, _split_w(W), _split_b(b))


def _mm2_body(o0, o1, o2, o3, w1_ref, b1_ref, w2_ref, b2_ref, o_ref):
    x = jnp.concatenate([o0[...], o1[...], o2[...], o3[...]], axis=1)
    t = jnp.dot(x, w1_ref[...], preferred_element_type=jnp.float32) + b1_ref[...]
    t = jnp.maximum(t, 0.0)
    o_ref[...] = (
        jnp.dot(t, w2_ref[0], preferred_element_type=jnp.float32) + b2_ref[0, 0]
    )


def _mm2(o_stacked, W1, b1, W2, b2):
    nb = N // _RB
    qspecs = [
        pl.BlockSpec((_RB, CW), (lambda qq: (lambda q, r: (qq * nb + r, 0)))(i))
        for i in range(NQ)
    ]
    return pl.pallas_call(
        _mm2_body,
        grid=(NQ, nb),
        in_specs=qspecs + [
            pl.BlockSpec((F, F), lambda q, r: (0, 0)),
            pl.BlockSpec((F,), lambda q, r: (0,)),
            pl.BlockSpec((1, F, CW), lambda q, r: (q, 0, 0)),
            pl.BlockSpec((1, 1, CW), lambda q, r: (q, 0, 0)),
        ],
        out_specs=pl.BlockSpec((_RB, CW), lambda q, r: (q * nb + r, 0)),
        out_shape=jax.ShapeDtypeStruct((NQ * N, CW), jnp.float32),
    )(o_stacked, o_stacked, o_stacked, o_stacked, W1, b1,
      _split_w(W2), _split_b(b2))


def _mm3_body(o0, o1, o2, o3, w_ref, b_ref, o_ref):
    x = jnp.concatenate([o0[...], o1[...], o2[...], o3[...]], axis=1)
    o_ref[...] = (
        jnp.dot(x, w_ref[...], preferred_element_type=jnp.float32) + b_ref[...]
    )


def _mm3(o_stacked, W, b):
    nb = N // _RB
    qspecs = [
        pl.BlockSpec((_RB, CW), (lambda qq: (lambda r: (qq * nb + r, 0)))(i))
        for i in range(NQ)
    ]
    return pl.pallas_call(
        _mm3_body,
        grid=(nb,),
        in_specs=qspecs + [
            pl.BlockSpec((F, NCLS), lambda r: (0, 0)),
            pl.BlockSpec((NCLS,), lambda r: (0,)),
        ],
        out_specs=pl.BlockSpec((_RB, NCLS), lambda r: (r, 0)),
        out_shape=jax.ShapeDtypeStruct((N, NCLS), jnp.float32),
    )(o_stacked, o_stacked, o_stacked, o_stacked, W, b)


def kernel(x, left_location, right_location, left_activity, right_activity,
           W11, b11, W12, b12, W21, b21, W22, b22):
    zi = jnp.zeros((PAD,), jnp.int32)
    zf = jnp.zeros((PAD,), jnp.float32)
    gl = jnp.concatenate([left_location, zi]).reshape(NS, CHUNKS, CH)
    gr = jnp.concatenate([right_location, zi]).reshape(NS, CHUNKS, CH)
    wl = jnp.concatenate([left_activity, zf]).reshape(NS, CHUNKS, CH)
    wr = jnp.concatenate([right_activity, zf]).reshape(NS, CHUNKS, CH)

    h1 = _mm1(x, W11, b11)
    e1 = _stage_n2e(h1, gl, gr, wl)
    o1 = _stage_e2n(e1, gr, gl, wr)
    h2 = _mm2(o1, W12, b12, W21, b21)
    e2 = _stage_n2e(h2, gl, gr, wl)
    o2 = _stage_e2n(e2, gr, gl, wr)
    return _mm3(o2, W22, b22)


# speedup vs baseline: 1.4543x; 1.4543x over previous
import functools

import numpy as np

import jax
import jax.numpy as jnp
from jax import lax
from jax.experimental import pallas as pl
from jax.experimental.pallas import tpu as pltpu
from jax.experimental.pallas import tpu_sc as plsc

N = 10000
E = 20000
NNZ = 320000
F = 128
CW = 32
NQ = 4
NCLS = 16

NC = 2
NS = 16
CH = 128
CHUNKS = 157
EPS = CH * CHUNKS
NNZ_PAD = NS * EPS
PAD = NNZ_PAD - NNZ

CWP = CW // 2


def _make_sc_stage(T, S):
    rows_s = (S // NS) & ~7
    last_rows = S - (NS - 1) * rows_s

    mesh = plsc.VectorSubcoreMesh(core_axis_name="c", subcore_axis_name="s")

    @functools.partial(
        pl.kernel,
        mesh=mesh,
        compiler_params=pltpu.CompilerParams(use_tc_tiling_on_sc=False),
        out_type=jax.ShapeDtypeStruct((NQ * S, CWP), jnp.uint32),
        scratch_types=[
            pltpu.VMEM((CHUNKS, CH), jnp.int32),
            pltpu.VMEM((CHUNKS, CH), jnp.int32),
            pltpu.VMEM((CHUNKS, CH), jnp.float32),
            pltpu.VMEM((CH, CWP), jnp.uint32),
            pltpu.VMEM((CH, CWP), jnp.uint32),
            pltpu.VMEM((CH, CWP), jnp.uint32),
            pltpu.VMEM((CH, CWP), jnp.uint32),
            pltpu.VMEM((CH, CW), jnp.float32),
            pltpu.VMEM((CH, CW), jnp.float32),
            pltpu.VMEM((CH, CW), jnp.float32),
            pltpu.VMEM((CH, CW), jnp.float32),
            pltpu.VMEM((16, CW), jnp.float32),
            pltpu.VMEM((16, CW), jnp.float32),
            pltpu.VMEM((16, CWP), jnp.uint32),
            pltpu.VMEM_SHARED((S, CW), jnp.float32),
            pltpu.SemaphoreType.DMA,
            pltpu.SemaphoreType.DMA,
            pltpu.SemaphoreType.DMA,
            pltpu.SemaphoreType.DMA,
            pltpu.SemaphoreType.DMA,
            pltpu.SemaphoreType.DMA,
            pltpu.SemaphoreType.DMA,
            pltpu.SemaphoreType.DMA,
        ],
    )
    def stage(table, gidx, sidx, w, out, gidx_v, sidx_v, w_v, rows0, rows1,
              rows2, rows3, srows0, srows1, srows2, srows3, zbuf, fbuf, wbuf,
              acc, g0, g1, g2, g3, s0, s1, s2, s3):
        c = lax.axis_index("c")
        s = lax.axis_index("s")

        pltpu.sync_copy(gidx.at[s], gidx_v)
        pltpu.sync_copy(sidx.at[s], sidx_v)
        pltpu.sync_copy(w.at[s], w_v)

        for r in range(16):
            for u in range(CW // 16):
                zbuf[r, pl.ds(u * 16, 16)] = jnp.zeros((16,), jnp.float32)
        base = s * rows_s
        n16 = jnp.where(s == NS - 1, last_rows // 16, rows_s // 16)

        slots = [(rows0, srows0, g0, s0), (rows1, srows1, g1, s1),
                 (rows2, srows2, g2, s2), (rows3, srows3, g3, s3)]

        def start_gather(i, slot):
            pltpu.async_copy(table.at[gidx_v.at[i]], slot[0], slot[2])

        def wait_gather(i, slot):
            pltpu.make_async_copy(table.at[gidx_v.at[i]], slot[0],
                                  slot[2]).wait()

        def start_scatter(i, slot):
            pltpu.async_copy(slot[1], acc.at[sidx_v.at[i]], slot[3], add=True)

        def wait_scatter(i, slot):
            pltpu.make_async_copy(slot[1], acc.at[sidx_v.at[i]],
                                  slot[3]).wait()

        def scale(i, slot):
            rows_buf, srows = slot[0], slot[1]

            def scale_body(g, carry2):
                wv = w_v[i, pl.ds(g * 16, 16)]
                for e in range(16):
                    ws = wv[e]
                    k = g * 16 + e
                    u = rows_buf[k, :]
                    a = lax.bitcast_convert_type(u << jnp.uint32(16),
                                                 jnp.float32)
                    b = lax.bitcast_convert_type(u & jnp.uint32(0xFFFF0000),
                                                 jnp.float32)
                    srows[k, pl.ds(0, 16)] = a * ws
                    srows[k, pl.ds(16, 16)] = b * ws
                return carry2

            lax.fori_loop(0, CH // 16, scale_body, 0)

        for p in range(2):
            q = 2 * c + p
            delta = (2 * c * T) if p == 0 else T

            def off_body(i, carry):
                for j in range(CH // 16):
                    sl = pl.ds(j * 16, 16)
                    gidx_v[i, sl] = gidx_v[i, sl] + delta
                return carry

            lax.fori_loop(0, CHUNKS, off_body, 0)

            def z_body(z, carry):
                pltpu.sync_copy(zbuf, acc.at[pl.ds(base + z * 16, 16)])
                return carry

            lax.fori_loop(0, n16, z_body, 0)
            plsc.subcore_barrier()

            start_gather(0, slots[0])
            start_gather(1, slots[1])
            for i0 in range(2):
                wait_gather(i0, slots[i0])
                scale(i0, slots[i0])
                start_gather(i0 + 2, slots[i0 + 2])
                start_scatter(i0, slots[i0])

            def quad_body(jj, carry):
                for k in range(4):
                    i = 2 + 4 * jj + k
                    sl = slots[(2 + k) % 4]
                    wait_gather(i, sl)
                    scale(i, sl)
                    wait_scatter(i - 2, slots[k])
                    start_gather(i + 2, slots[k])
                    start_scatter(i, sl)
                return carry

            lax.fori_loop(0, (CHUNKS - 5) // 4, quad_body, 0)
            i_e = CHUNKS - 3
            wait_gather(i_e, slots[2])
            scale(i_e, slots[2])
            wait_scatter(i_e - 2, slots[0])
            start_gather(i_e + 2, slots[0])
            start_scatter(i_e, slots[2])
            wait_gather(i_e + 1, slots[3])
            scale(i_e + 1, slots[3])
            wait_scatter(i_e - 1, slots[1])
            start_scatter(i_e + 1, slots[3])
            wait_gather(i_e + 2, slots[0])
            scale(i_e + 2, slots[0])
            wait_scatter(i_e, slots[2])
            start_scatter(i_e + 2, slots[0])
            wait_scatter(i_e + 1, slots[3])
            wait_scatter(i_e + 2, slots[0])
            plsc.subcore_barrier()

            def wb_body(z, carry):
                r0 = base + z * 16
                pltpu.sync_copy(acc.at[pl.ds(r0, 16)], fbuf)
                for r in range(16):
                    au = lax.bitcast_convert_type(fbuf[r, pl.ds(0, 16)],
                                                  jnp.uint32)
                    bu = lax.bitcast_convert_type(fbuf[r, pl.ds(16, 16)],
                                                  jnp.uint32)
                    au = au + jnp.uint32(0x7FFF) + ((au >> jnp.uint32(16))
                                                    & jnp.uint32(1))
                    bu = bu + jnp.uint32(0x7FFF) + ((bu >> jnp.uint32(16))
                                                    & jnp.uint32(1))
                    wbuf[r, :] = (au >> jnp.uint32(16)) | (
                        bu & jnp.uint32(0xFFFF0000))
                pltpu.sync_copy(wbuf, out.at[pl.ds(q * S + r0, 16)])
                return carry

            lax.fori_loop(0, n16, wb_body, 0)
            plsc.subcore_barrier()

    return stage


_raw_stage_n2e = _make_sc_stage(N, E)
_raw_stage_e2n = _make_sc_stage(E, N)


def _hbm(x):
    return pltpu.with_memory_space_constraint(x, pltpu.MemorySpace.HBM)


def _stage_n2e(table, gidx, sidx, w):
    return _raw_stage_n2e(_hbm(table), _hbm(gidx), _hbm(sidx), _hbm(w))


def _stage_e2n(table, gidx, sidx, w):
    return _raw_stage_e2n(_hbm(table), _hbm(gidx), _hbm(sidx), _hbm(w))


_RB = 2000


def _split_w(W):
    k = W.shape[1] // NQ
    return jnp.stack([W[:, i * k:(i + 1) * k] for i in range(NQ)])


def _split_b(b):
    return b.reshape(NQ, 1, b.shape[0] // NQ)


def _pack_tc(t):
    lo = jax.lax.bitcast_convert_type(
        t[:, :CWP].astype(jnp.bfloat16), jnp.uint16).astype(jnp.uint32)
    hi = jax.lax.bitcast_convert_type(
        t[:, CWP:].astype(jnp.bfloat16), jnp.uint16).astype(jnp.uint32)
    return lo | (hi << 16)


def _unpack_tc(xq):
    lo = jax.lax.bitcast_convert_type(
        (xq & jnp.uint32(0xFFFF)).astype(jnp.uint16),
        jnp.bfloat16).astype(jnp.float32)
    hi = jax.lax.bitcast_convert_type(
        (xq >> jnp.uint32(16)).astype(jnp.uint16),
        jnp.bfloat16).astype(jnp.float32)
    return jnp.concatenate([lo, hi], axis=1)


def _mm1_body(x_ref, w_ref, b_ref, o_ref):
    o_ref[...] = _pack_tc(
        jnp.dot(x_ref[...], w_ref[0], preferred_element_type=jnp.float32)
        + b_ref[0, 0]
    )


def _mm1(x, W, b):
    nb = N // _RB
    return pl.pallas_call(
        _mm1_body,
        grid=(NQ, nb),
        in_specs=[
            pl.BlockSpec((_RB, F), lambda q, r: (r, 0)),
            pl.BlockSpec((1, F, CW), lambda q, r: (q, 0, 0)),
            pl.BlockSpec((1, 1, CW), lambda q, r: (q, 0, 0)),
        ],
        out_specs=pl.BlockSpec((_RB, CWP), lambda q, r: (q * nb + r, 0)),
        out_shape=jax.ShapeDtypeStruct((NQ * N, CWP), jnp.uint32),
    )(x, _split_w(W), _split_b(b))


def _mm2_body(o0, o1, o2, o3, w1_ref, b1_ref, w2_ref, b2_ref, o_ref):
    x = jnp.concatenate(
        [_unpack_tc(o0[...]), _unpack_tc(o1[...]),
         _unpack_tc(o2[...]), _unpack_tc(o3[...])], axis=1)
    t = jnp.dot(x, w1_ref[...], preferred_element_type=jnp.float32) + b1_ref[...]
    t = jnp.maximum(t, 0.0)
    o_ref[...] = _pack_tc(
        jnp.dot(t, w2_ref[0], preferred_element_type=jnp.float32) + b2_ref[0, 0]
    )


def _mm2(o_stacked, W1, b1, W2, b2):
    nb = N // _RB
    qspecs = [
        pl.BlockSpec((_RB, CWP), (lambda qq: (lambda q, r: (qq * nb + r, 0)))(i))
        for i in range(NQ)
    ]
    return pl.pallas_call(
        _mm2_body,
        grid=(NQ, nb),
        in_specs=qspecs + [
            pl.BlockSpec((F, F), lambda q, r: (0, 0)),
            pl.BlockSpec((F,), lambda q, r: (0,)),
            pl.BlockSpec((1, F, CW), lambda q, r: (q, 0, 0)),
            pl.BlockSpec((1, 1, CW), lambda q, r: (q, 0, 0)),
        ],
        out_specs=pl.BlockSpec((_RB, CWP), lambda q, r: (q * nb + r, 0)),
        out_shape=jax.ShapeDtypeStruct((NQ * N, CWP), jnp.uint32),
    )(o_stacked, o_stacked, o_stacked, o_stacked, W1, b1,
      _split_w(W2), _split_b(b2))


def _mm3_body(o0, o1, o2, o3, w_ref, b_ref, o_ref):
    x = jnp.concatenate(
        [_unpack_tc(o0[...]), _unpack_tc(o1[...]),
         _unpack_tc(o2[...]), _unpack_tc(o3[...])], axis=1)
    o_ref[...] = (
        jnp.dot(x, w_ref[...], preferred_element_type=jnp.float32) + b_ref[...]
    )


def _mm3(o_stacked, W, b):
    nb = N // _RB
    qspecs = [
        pl.BlockSpec((_RB, CWP), (lambda qq: (lambda r: (qq * nb + r, 0)))(i))
        for i in range(NQ)
    ]
    return pl.pallas_call(
        _mm3_body,
        grid=(nb,),
        in_specs=qspecs + [
            pl.BlockSpec((F, NCLS), lambda r: (0, 0)),
            pl.BlockSpec((NCLS,), lambda r: (0,)),
        ],
        out_specs=pl.BlockSpec((_RB, NCLS), lambda r: (r, 0)),
        out_shape=jax.ShapeDtypeStruct((N, NCLS), jnp.float32),
    )(o_stacked, o_stacked, o_stacked, o_stacked, W, b)


def kernel(x, left_location, right_location, left_activity, right_activity,
           W11, b11, W12, b12, W21, b21, W22, b22):
    zi = jnp.zeros((PAD,), jnp.int32)
    zf = jnp.zeros((PAD,), jnp.float32)
    gl = jnp.concatenate([left_location, zi]).reshape(NS, CHUNKS, CH)
    gr = jnp.concatenate([right_location, zi]).reshape(NS, CHUNKS, CH)
    wl = jnp.concatenate([left_activity, zf]).reshape(NS, CHUNKS, CH)
    wr = jnp.concatenate([right_activity, zf]).reshape(NS, CHUNKS, CH)

    h1 = _mm1(x, W11, b11)
    e1 = _stage_n2e(h1, gl, gr, wl)
    o1 = _stage_e2n(e1, gr, gl, wr)
    h2 = _mm2(o1, W12, b12, W21, b21)
    e2 = _stage_n2e(h2, gl, gr, wl)
    o2 = _stage_e2n(e2, gr, gl, wr)
    return _mm3(o2, W22, b22)


# speedup vs baseline: 1.4775x; 1.0160x over previous
import functools

import numpy as np

import jax
import jax.numpy as jnp
from jax import lax
from jax.experimental import pallas as pl
from jax.experimental.pallas import tpu as pltpu
from jax.experimental.pallas import tpu_sc as plsc

N = 10000
E = 20000
NNZ = 320000
F = 128
CW = 32
NQ = 4
NCLS = 16

NC = 2
NS = 16
CH = 128
CHUNKS = 157
EPS = CH * CHUNKS
NNZ_PAD = NS * EPS
PAD = NNZ_PAD - NNZ

CWP = CW // 2


def _make_sc_stage(T, S):
    rows_s = (S // NS) & ~7
    last_rows = S - (NS - 1) * rows_s

    mesh = plsc.VectorSubcoreMesh(core_axis_name="c", subcore_axis_name="s")

    @functools.partial(
        pl.kernel,
        mesh=mesh,
        compiler_params=pltpu.CompilerParams(use_tc_tiling_on_sc=False),
        out_type=jax.ShapeDtypeStruct((NQ * S, CWP), jnp.uint32),
        scratch_types=[
            pltpu.VMEM((CHUNKS, CH), jnp.int32),
            pltpu.VMEM((CHUNKS, CH), jnp.int32),
            pltpu.VMEM((CHUNKS, CH), jnp.float32),
            pltpu.VMEM((CH, CWP), jnp.uint32),
            pltpu.VMEM((CH, CWP), jnp.uint32),
            pltpu.VMEM((CH, CWP), jnp.uint32),
            pltpu.VMEM((CH, CWP), jnp.uint32),
            pltpu.VMEM((CH, CW), jnp.float32),
            pltpu.VMEM((CH, CW), jnp.float32),
            pltpu.VMEM((CH, CW), jnp.float32),
            pltpu.VMEM((CH, CW), jnp.float32),
            pltpu.VMEM((16, CW), jnp.float32),
            pltpu.VMEM((16, CW), jnp.float32),
            pltpu.VMEM((16, CWP), jnp.uint32),
            pltpu.VMEM_SHARED((S, CW), jnp.float32),
            pltpu.SemaphoreType.DMA,
            pltpu.SemaphoreType.DMA,
            pltpu.SemaphoreType.DMA,
            pltpu.SemaphoreType.DMA,
            pltpu.SemaphoreType.DMA,
            pltpu.SemaphoreType.DMA,
            pltpu.SemaphoreType.DMA,
            pltpu.SemaphoreType.DMA,
        ],
    )
    def stage(table, gidx, sidx, w, out, gidx_v, sidx_v, w_v, rows0, rows1,
              rows2, rows3, srows0, srows1, srows2, srows3, zbuf, fbuf, wbuf,
              acc, g0, g1, g2, g3, s0, s1, s2, s3):
        c = lax.axis_index("c")
        s = lax.axis_index("s")

        pltpu.sync_copy(gidx.at[s], gidx_v)
        pltpu.sync_copy(sidx.at[s], sidx_v)
        pltpu.sync_copy(w.at[s], w_v)

        for r in range(16):
            for u in range(CW // 16):
                zbuf[r, pl.ds(u * 16, 16)] = jnp.zeros((16,), jnp.float32)
        base = s * rows_s
        n16 = jnp.where(s == NS - 1, last_rows // 16, rows_s // 16)

        slots = [(rows0, srows0, g0, s0), (rows1, srows1, g1, s1),
                 (rows2, srows2, g2, s2), (rows3, srows3, g3, s3)]

        def start_gather(i, slot):
            pltpu.async_copy(table.at[gidx_v.at[i]], slot[0], slot[2])

        def wait_gather(i, slot):
            pltpu.make_async_copy(table.at[gidx_v.at[i]], slot[0],
                                  slot[2]).wait()

        def start_scatter(i, slot):
            pltpu.async_copy(slot[1], acc.at[sidx_v.at[i]], slot[3], add=True)

        def wait_scatter(i, slot):
            pltpu.make_async_copy(slot[1], acc.at[sidx_v.at[i]],
                                  slot[3]).wait()

        def scale(i, slot):
            rows_buf, srows = slot[0], slot[1]

            def scale_body(g, carry2):
                wv = w_v[i, pl.ds(g * 16, 16)]
                for e in range(16):
                    ws = wv[e]
                    k = g * 16 + e
                    u = rows_buf[k, :]
                    a = lax.bitcast_convert_type(u << jnp.uint32(16),
                                                 jnp.float32)
                    b = lax.bitcast_convert_type(u & jnp.uint32(0xFFFF0000),
                                                 jnp.float32)
                    srows[k, pl.ds(0, 16)] = a * ws
                    srows[k, pl.ds(16, 16)] = b * ws
                return carry2

            lax.fori_loop(0, CH // 16, scale_body, 0)

        for p in range(2):
            q = 2 * c + p
            delta = (2 * c * T) if p == 0 else T

            def off_body(i, carry):
                for j in range(CH // 16):
                    sl = pl.ds(j * 16, 16)
                    gidx_v[i, sl] = gidx_v[i, sl] + delta
                return carry

            lax.fori_loop(0, CHUNKS, off_body, 0)

            def z_body(z, carry):
                pltpu.sync_copy(zbuf, acc.at[pl.ds(base + z * 16, 16)])
                return carry

            lax.fori_loop(0, n16, z_body, 0)
            plsc.subcore_barrier()

            start_gather(0, slots[0])
            start_gather(1, slots[1])
            for i0 in range(2):
                start_gather(i0 + 2, slots[i0 + 2])
                wait_gather(i0, slots[i0])
                scale(i0, slots[i0])
                start_scatter(i0, slots[i0])

            def quad_body(jj, carry):
                for k in range(4):
                    i = 2 + 4 * jj + k
                    sl = slots[(2 + k) % 4]
                    wait_scatter(i - 2, slots[k])
                    start_gather(i + 2, slots[k])
                    wait_gather(i, sl)
                    scale(i, sl)
                    start_scatter(i, sl)
                return carry

            lax.fori_loop(0, (CHUNKS - 5) // 4, quad_body, 0)
            i_e = CHUNKS - 3
            wait_scatter(i_e - 2, slots[0])
            start_gather(i_e + 2, slots[0])
            wait_gather(i_e, slots[2])
            scale(i_e, slots[2])
            start_scatter(i_e, slots[2])
            wait_gather(i_e + 1, slots[3])
            scale(i_e + 1, slots[3])
            wait_scatter(i_e - 1, slots[1])
            start_scatter(i_e + 1, slots[3])
            wait_gather(i_e + 2, slots[0])
            scale(i_e + 2, slots[0])
            wait_scatter(i_e, slots[2])
            start_scatter(i_e + 2, slots[0])
            wait_scatter(i_e + 1, slots[3])
            wait_scatter(i_e + 2, slots[0])
            plsc.subcore_barrier()

            def wb_body(z, carry):
                r0 = base + z * 16
                pltpu.sync_copy(acc.at[pl.ds(r0, 16)], fbuf)
                for r in range(16):
                    au = lax.bitcast_convert_type(fbuf[r, pl.ds(0, 16)],
                                                  jnp.uint32)
                    bu = lax.bitcast_convert_type(fbuf[r, pl.ds(16, 16)],
                                                  jnp.uint32)
                    au = au + jnp.uint32(0x7FFF) + ((au >> jnp.uint32(16))
                                                    & jnp.uint32(1))
                    bu = bu + jnp.uint32(0x7FFF) + ((bu >> jnp.uint32(16))
                                                    & jnp.uint32(1))
                    wbuf[r, :] = (au >> jnp.uint32(16)) | (
                        bu & jnp.uint32(0xFFFF0000))
                pltpu.sync_copy(wbuf, out.at[pl.ds(q * S + r0, 16)])
                return carry

            lax.fori_loop(0, n16, wb_body, 0)
            plsc.subcore_barrier()

    return stage


_raw_stage_n2e = _make_sc_stage(N, E)
_raw_stage_e2n = _make_sc_stage(E, N)


def _hbm(x):
    return pltpu.with_memory_space_constraint(x, pltpu.MemorySpace.HBM)


def _stage_n2e(table, gidx, sidx, w):
    return _raw_stage_n2e(_hbm(table), _hbm(gidx), _hbm(sidx), _hbm(w))


def _stage_e2n(table, gidx, sidx, w):
    return _raw_stage_e2n(_hbm(table), _hbm(gidx), _hbm(sidx), _hbm(w))


_RB = 2000


def _split_w(W):
    k = W.shape[1] // NQ
    return jnp.stack([W[:, i * k:(i + 1) * k] for i in range(NQ)])


def _split_b(b):
    return b.reshape(NQ, 1, b.shape[0] // NQ)


def _pack_tc(t):
    lo = jax.lax.bitcast_convert_type(
        t[:, :CWP].astype(jnp.bfloat16), jnp.uint16).astype(jnp.uint32)
    hi = jax.lax.bitcast_convert_type(
        t[:, CWP:].astype(jnp.bfloat16), jnp.uint16).astype(jnp.uint32)
    return lo | (hi << 16)


def _unpack_tc(xq):
    lo = jax.lax.bitcast_convert_type(
        (xq & jnp.uint32(0xFFFF)).astype(jnp.uint16),
        jnp.bfloat16).astype(jnp.float32)
    hi = jax.lax.bitcast_convert_type(
        (xq >> jnp.uint32(16)).astype(jnp.uint16),
        jnp.bfloat16).astype(jnp.float32)
    return jnp.concatenate([lo, hi], axis=1)


def _mm1_body(x_ref, w_ref, b_ref, o_ref):
    o_ref[...] = _pack_tc(
        jnp.dot(x_ref[...], w_ref[0], preferred_element_type=jnp.float32)
        + b_ref[0, 0]
    )


def _mm1(x, W, b):
    nb = N // _RB
    return pl.pallas_call(
        _mm1_body,
        grid=(NQ, nb),
        in_specs=[
            pl.BlockSpec((_RB, F), lambda q, r: (r, 0)),
            pl.BlockSpec((1, F, CW), lambda q, r: (q, 0, 0)),
            pl.BlockSpec((1, 1, CW), lambda q, r: (q, 0, 0)),
        ],
        out_specs=pl.BlockSpec((_RB, CWP), lambda q, r: (q * nb + r, 0)),
        out_shape=jax.ShapeDtypeStruct((NQ * N, CWP), jnp.uint32),
    )(x, _split_w(W), _split_b(b))


def _mm2_body(o0, o1, o2, o3, w1_ref, b1_ref, w2_ref, b2_ref, o_ref):
    x = jnp.concatenate(
        [_unpack_tc(o0[...]), _unpack_tc(o1[...]),
         _unpack_tc(o2[...]), _unpack_tc(o3[...])], axis=1)
    t = jnp.dot(x, w1_ref[...], preferred_element_type=jnp.float32) + b1_ref[...]
    t = jnp.maximum(t, 0.0)
    o_ref[...] = _pack_tc(
        jnp.dot(t, w2_ref[0], preferred_element_type=jnp.float32) + b2_ref[0, 0]
    )


def _mm2(o_stacked, W1, b1, W2, b2):
    nb = N // _RB
    qspecs = [
        pl.BlockSpec((_RB, CWP), (lambda qq: (lambda q, r: (qq * nb + r, 0)))(i))
        for i in range(NQ)
    ]
    return pl.pallas_call(
        _mm2_body,
        grid=(NQ, nb),
        in_specs=qspecs + [
            pl.BlockSpec((F, F), lambda q, r: (0, 0)),
            pl.BlockSpec((F,), lambda q, r: (0,)),
            pl.BlockSpec((1, F, CW), lambda q, r: (q, 0, 0)),
            pl.BlockSpec((1, 1, CW), lambda q, r: (q, 0, 0)),
        ],
        out_specs=pl.BlockSpec((_RB, CWP), lambda q, r: (q * nb + r, 0)),
        out_shape=jax.ShapeDtypeStruct((NQ * N, CWP), jnp.uint32),
    )(o_stacked, o_stacked, o_stacked, o_stacked, W1, b1,
      _split_w(W2), _split_b(b2))


def _mm3_body(o0, o1, o2, o3, w_ref, b_ref, o_ref):
    x = jnp.concatenate(
        [_unpack_tc(o0[...]), _unpack_tc(o1[...]),
         _unpack_tc(o2[...]), _unpack_tc(o3[...])], axis=1)
    o_ref[...] = (
        jnp.dot(x, w_ref[...], preferred_element_type=jnp.float32) + b_ref[...]
    )


def _mm3(o_stacked, W, b):
    nb = N // _RB
    qspecs = [
        pl.BlockSpec((_RB, CWP), (lambda qq: (lambda r: (qq * nb + r, 0)))(i))
        for i in range(NQ)
    ]
    return pl.pallas_call(
        _mm3_body,
        grid=(nb,),
        in_specs=qspecs + [
            pl.BlockSpec((F, NCLS), lambda r: (0, 0)),
            pl.BlockSpec((NCLS,), lambda r: (0,)),
        ],
        out_specs=pl.BlockSpec((_RB, NCLS), lambda r: (r, 0)),
        out_shape=jax.ShapeDtypeStruct((N, NCLS), jnp.float32),
    )(o_stacked, o_stacked, o_stacked, o_stacked, W, b)


def kernel(x, left_location, right_location, left_activity, right_activity,
           W11, b11, W12, b12, W21, b21, W22, b22):
    zi = jnp.zeros((PAD,), jnp.int32)
    zf = jnp.zeros((PAD,), jnp.float32)
    gl = jnp.concatenate([left_location, zi]).reshape(NS, CHUNKS, CH)
    gr = jnp.concatenate([right_location, zi]).reshape(NS, CHUNKS, CH)
    wl = jnp.concatenate([left_activity, zf]).reshape(NS, CHUNKS, CH)
    wr = jnp.concatenate([right_activity, zf]).reshape(NS, CHUNKS, CH)

    h1 = _mm1(x, W11, b11)
    e1 = _stage_n2e(h1, gl, gr, wl)
    o1 = _stage_e2n(e1, gr, gl, wr)
    h2 = _mm2(o1, W12, b12, W21, b21)
    e2 = _stage_n2e(h2, gl, gr, wl)
    o2 = _stage_e2n(e2, gr, gl, wr)
    return _mm3(o2, W22, b22)


# speedup vs baseline: 1.6985x; 1.1496x over previous
import functools

import jax
import jax.numpy as jnp
from jax import lax
from jax.experimental import pallas as pl
from jax.experimental.pallas import tpu as pltpu
from jax.experimental.pallas import tpu_sc as plsc

N = 10000
E = 20000
NNZ = 320000
F = 128
CW = 32
NQ = 4
NCLS = 16

NC = 2
NS = 16
CH = 128
CHUNKS = 157
EPS = CH * CHUNKS
NNZ_PAD = NS * EPS
PAD = NNZ_PAD - NNZ


def _make_sc_stage(T, S):
    rows_s = (S // NS) & ~7
    last_rows = S - (NS - 1) * rows_s

    mesh = plsc.VectorSubcoreMesh(core_axis_name="c", subcore_axis_name="s")

    @functools.partial(
        pl.kernel,
        mesh=mesh,
        compiler_params=pltpu.CompilerParams(use_tc_tiling_on_sc=False),
        out_type=jax.ShapeDtypeStruct((NQ * S, CW), jnp.float32),
        scratch_types=[
            pltpu.VMEM((CHUNKS, CH), jnp.int32),
            pltpu.VMEM((CHUNKS, CH), jnp.int32),
            pltpu.VMEM((CHUNKS, CH), jnp.float32),
            pltpu.VMEM((CH, CW), jnp.float32),
            pltpu.VMEM((CH, CW), jnp.float32),
            pltpu.VMEM((CH, CW), jnp.float32),
            pltpu.VMEM((CH, CW), jnp.float32),
            pltpu.VMEM((16, CW), jnp.float32),
            pltpu.VMEM_SHARED((S, CW), jnp.float32),
            pltpu.SemaphoreType.DMA,
            pltpu.SemaphoreType.DMA,
            pltpu.SemaphoreType.DMA,
            pltpu.SemaphoreType.DMA,
            pltpu.SemaphoreType.DMA,
            pltpu.SemaphoreType.DMA,
            pltpu.SemaphoreType.DMA,
            pltpu.SemaphoreType.DMA,
        ],
    )
    def stage(table, gidx, sidx, w, out, gidx_v, sidx_v, w_v, rows0, rows1,
              rows2, rows3, zbuf, acc, g0, g1, g2, g3, s0, s1, s2, s3):
        c = lax.axis_index("c")
        s = lax.axis_index("s")

        pltpu.sync_copy(gidx.at[s], gidx_v)
        pltpu.sync_copy(sidx.at[s], sidx_v)
        pltpu.sync_copy(w.at[s], w_v)

        for r in range(16):
            for u in range(CW // 16):
                zbuf[r, pl.ds(u * 16, 16)] = jnp.zeros((16,), jnp.float32)
        base = s * rows_s
        n16 = jnp.where(s == NS - 1, last_rows // 16, rows_s // 16)

        slots = [(rows0, g0, s0), (rows1, g1, s1), (rows2, g2, s2),
                 (rows3, g3, s3)]

        def start_gather(i, slot):
            pltpu.async_copy(table.at[gidx_v.at[i]], slot[0], slot[1])

        def wait_gather(i, slot):
            pltpu.make_async_copy(table.at[gidx_v.at[i]], slot[0],
                                  slot[1]).wait()

        def start_scatter(i, slot):
            pltpu.async_copy(slot[0], acc.at[sidx_v.at[i]], slot[2], add=True)

        def wait_scatter(i, slot):
            pltpu.make_async_copy(slot[0], acc.at[sidx_v.at[i]],
                                  slot[2]).wait()

        def scale(i, slot):
            rows_buf = slot[0]

            def scale_body(g, carry2):
                wv = w_v[i, pl.ds(g * 16, 16)]
                for e in range(16):
                    ws = wv[e]
                    k = g * 16 + e
                    for u in range(CW // 16):
                        sl = pl.ds(u * 16, 16)
                        rows_buf[k, sl] = rows_buf[k, sl] * ws
                return carry2

            lax.fori_loop(0, CH // 16, scale_body, 0)

        for p in range(2):
            q = 2 * c + p
            delta = (2 * c * T) if p == 0 else T

            def off_body(i, carry):
                for j in range(CH // 16):
                    sl = pl.ds(j * 16, 16)
                    gidx_v[i, sl] = gidx_v[i, sl] + delta
                return carry

            lax.fori_loop(0, CHUNKS, off_body, 0)

            def z_body(z, carry):
                pltpu.sync_copy(zbuf, acc.at[pl.ds(base + z * 16, 16)])
                return carry

            lax.fori_loop(0, n16, z_body, 0)
            plsc.subcore_barrier()

            start_gather(0, slots[0])
            start_gather(1, slots[1])
            for i0 in range(2):
                start_gather(i0 + 2, slots[i0 + 2])
                wait_gather(i0, slots[i0])
                scale(i0, slots[i0])
                start_scatter(i0, slots[i0])

            def quad_body(jj, carry):
                for k in range(4):
                    i = 2 + 4 * jj + k
                    sl = slots[(2 + k) % 4]
                    wait_scatter(i - 2, slots[k])
                    start_gather(i + 2, slots[k])
                    wait_gather(i, sl)
                    scale(i, sl)
                    start_scatter(i, sl)
                return carry

            lax.fori_loop(0, (CHUNKS - 5) // 4, quad_body, 0)
            i_e = CHUNKS - 3
            wait_scatter(i_e - 2, slots[0])
            start_gather(i_e + 2, slots[0])
            wait_gather(i_e, slots[2])
            scale(i_e, slots[2])
            start_scatter(i_e, slots[2])
            wait_gather(i_e + 1, slots[3])
            scale(i_e + 1, slots[3])
            wait_scatter(i_e - 1, slots[1])
            start_scatter(i_e + 1, slots[3])
            wait_gather(i_e + 2, slots[0])
            scale(i_e + 2, slots[0])
            wait_scatter(i_e, slots[2])
            start_scatter(i_e + 2, slots[0])
            wait_scatter(i_e + 1, slots[3])
            wait_scatter(i_e + 2, slots[0])
            plsc.subcore_barrier()

            def wb_body(z, carry):
                r0 = base + z * 16
                pltpu.sync_copy(acc.at[pl.ds(r0, 16)],
                                out.at[pl.ds(q * S + r0, 16)])
                return carry

            lax.fori_loop(0, n16, wb_body, 0)
            plsc.subcore_barrier()

    return stage


_raw_stage_n2e = _make_sc_stage(N, E)
_raw_stage_e2n = _make_sc_stage(E, N)


def _hbm(x):
    return pltpu.with_memory_space_constraint(x, pltpu.MemorySpace.HBM)


def _stage_n2e(table, gidx, sidx, w):
    return _raw_stage_n2e(_hbm(table), _hbm(gidx), _hbm(sidx), _hbm(w))


def _stage_e2n(table, gidx, sidx, w):
    return _raw_stage_e2n(_hbm(table), _hbm(gidx), _hbm(sidx), _hbm(w))


_RB = 1000


def _split_w(W):
    k = W.shape[1] // NQ
    return jnp.stack([W[:, i * k:(i + 1) * k] for i in range(NQ)])


def _split_b(b):
    return b.reshape(NQ, 1, b.shape[0] // NQ)


def _mm1_body(x_ref, w_ref, b_ref, o_ref):
    o_ref[...] = (
        jnp.dot(x_ref[...], w_ref[0], preferred_element_type=jnp.float32)
        + b_ref[0, 0]
    )


def _mm1(x, W, b):
    nb = N // _RB
    return pl.pallas_call(
        _mm1_body,
        grid=(NQ, nb),
        in_specs=[
            pl.BlockSpec((_RB, F), lambda q, r: (r, 0)),
            pl.BlockSpec((1, F, CW), lambda q, r: (q, 0, 0)),
            pl.BlockSpec((1, 1, CW), lambda q, r: (q, 0, 0)),
        ],
        out_specs=pl.BlockSpec((_RB, CW), lambda q, r: (q * nb + r, 0)),
        out_shape=jax.ShapeDtypeStruct((NQ * N, CW), jnp.float32),
    )(x, _split_w(W), _split_b(b))


def _mm2_body(o0, o1, o2, o3, w1_ref, b1_ref, w2_ref, b2_ref, o_ref):
    x = jnp.concatenate([o0[...], o1[...], o2[...], o3[...]], axis=1)
    t = jnp.dot(x, w1_ref[...], preferred_element_type=jnp.float32) + b1_ref[...]
    t = jnp.maximum(t, 0.0)
    o_ref[...] = (
        jnp.dot(t, w2_ref[0], preferred_element_type=jnp.float32) + b2_ref[0, 0]
    )


def _mm2(o_stacked, W1, b1, W2, b2):
    nb = N // _RB
    qspecs = [
        pl.BlockSpec((_RB, CW), (lambda qq: (lambda q, r: (qq * nb + r, 0)))(i))
        for i in range(NQ)
    ]
    return pl.pallas_call(
        _mm2_body,
        grid=(NQ, nb),
        in_specs=qspecs + [
            pl.BlockSpec((F, F), lambda q, r: (0, 0)),
            pl.BlockSpec((F,), lambda q, r: (0,)),
            pl.BlockSpec((1, F, CW), lambda q, r: (q, 0, 0)),
            pl.BlockSpec((1, 1, CW), lambda q, r: (q, 0, 0)),
        ],
        out_specs=pl.BlockSpec((_RB, CW), lambda q, r: (q * nb + r, 0)),
        out_shape=jax.ShapeDtypeStruct((NQ * N, CW), jnp.float32),
    )(o_stacked, o_stacked, o_stacked, o_stacked, W1, b1,
      _split_w(W2), _split_b(b2))


def _mm3_body(o0, o1, o2, o3, w_ref, b_ref, o_ref):
    x = jnp.concatenate([o0[...], o1[...], o2[...], o3[...]], axis=1)
    o_ref[...] = (
        jnp.dot(x, w_ref[...], preferred_element_type=jnp.float32) + b_ref[...]
    )


def _mm3(o_stacked, W, b):
    nb = N // _RB
    qspecs = [
        pl.BlockSpec((_RB, CW), (lambda qq: (lambda r: (qq * nb + r, 0)))(i))
        for i in range(NQ)
    ]
    return pl.pallas_call(
        _mm3_body,
        grid=(nb,),
        in_specs=qspecs + [
            pl.BlockSpec((F, NCLS), lambda r: (0, 0)),
            pl.BlockSpec((NCLS,), lambda r: (0,)),
        ],
        out_specs=pl.BlockSpec((_RB, NCLS), lambda r: (r, 0)),
        out_shape=jax.ShapeDtypeStruct((N, NCLS), jnp.float32),
    )(o_stacked, o_stacked, o_stacked, o_stacked, W, b)


def kernel(x, left_location, right_location, left_activity, right_activity,
           W11, b11, W12, b12, W21, b21, W22, b22):
    zi = jnp.zeros((PAD,), jnp.int32)
    zf = jnp.zeros((PAD,), jnp.float32)
    gl = jnp.concatenate([left_location, zi]).reshape(NS, CHUNKS, CH)
    gr = jnp.concatenate([right_location, zi]).reshape(NS, CHUNKS, CH)
    wl = jnp.concatenate([left_activity, zf]).reshape(NS, CHUNKS, CH)
    wr = jnp.concatenate([right_activity, zf]).reshape(NS, CHUNKS, CH)

    h1 = _mm1(x, W11, b11)
    e1 = _stage_n2e(h1, gl, gr, wl)
    o1 = _stage_e2n(e1, gr, gl, wr)
    h2 = _mm2(o1, W12, b12, W21, b21)
    e2 = _stage_n2e(h2, gl, gr, wl)
    o2 = _stage_e2n(e2, gr, gl, wr)
    return _mm3(o2, W22, b22)
